# baseline (speedup 1.0000x reference)
import functools

import numpy as np
import jax
import jax.numpy as jnp
from jax import lax
from jax.experimental import pallas as pl
from jax.experimental.pallas import tpu as pltpu
from jax.experimental.pallas import tpu_sc as plsc

D_MODEL = 64
NUM_CORES = 2
NUM_SUBCORES = 16
NUM_WORKERS = NUM_CORES * NUM_SUBCORES
LANES = 16
SCALE = float(np.sqrt(D_MODEL))

V_TOTAL = 1_000_000
V_MAIN = (V_TOTAL // 128) * 128
V_TAIL = V_TOTAL - V_MAIN
BLK_COLS = 256
N_BLOCKS = V_MAIN // BLK_COLS
BLOCKS_PER_W = N_BLOCKS // NUM_WORKERS
N_EXTRA = N_BLOCKS - BLOCKS_PER_W * NUM_WORKERS


def _mesh():
    return plsc.VectorSubcoreMesh(core_axis_name="c", subcore_axis_name="s")


@functools.lru_cache(maxsize=None)
def _make_transpose():
    @functools.partial(
        pl.kernel,
        out_type=jax.ShapeDtypeStruct((V_TOTAL // 2, 128), jnp.float32),
        mesh=_mesh(),
        compiler_params=pltpu.CompilerParams(
            use_tc_tiling_on_sc=True, needs_layout_passes=False),
        scratch_types=[
            pltpu.VMEM((D_MODEL, BLK_COLS), jnp.float32),
            pltpu.VMEM((D_MODEL, BLK_COLS), jnp.float32),
            pltpu.VMEM((BLK_COLS // 2, 128), jnp.float32),
            pltpu.VMEM((BLK_COLS // 2, 128), jnp.float32),
            pltpu.VMEM((32, 128), jnp.float32),
            pltpu.SemaphoreType.DMA,
            pltpu.SemaphoreType.DMA,
            pltpu.SemaphoreType.DMA,
            pltpu.SemaphoreType.DMA,
        ],
    )
    def k1(tt_hbm, tail_hbm, out_hbm, in_a, in_b, o_a, o_b, tbuf,
           si_a, si_b, so_a, so_b):
        wid = lax.axis_index("s") * NUM_CORES + lax.axis_index("c")
        jb0 = wid * BLOCKS_PER_W

        fvec = [
            lax.iota(jnp.int32, 16) + (c0 * 16) for c0 in range(D_MODEL // 16)
        ]

        def in_dma(jb, buf, sem):
            col = pl.multiple_of(jb * BLK_COLS, BLK_COLS)
            return pltpu.make_async_copy(
                tt_hbm.at[:, pl.ds(col, BLK_COLS)], buf, sem)

        def out_dma(jb, buf, sem):
            row = pl.multiple_of(jb * (BLK_COLS // 2), BLK_COLS // 2)
            return pltpu.make_async_copy(
                buf, out_hbm.at[pl.ds(row, BLK_COLS // 2), :], sem)

        def transpose(in_v, out_v):
            def row_body(r, carry):
                c_even = jnp.broadcast_to(2 * r, (16,)).astype(jnp.int32)
                c_odd = c_even + 1
                for c0 in range(4):
                    out_v[r, pl.ds(c0 * 16, 16)] = plsc.load_gather(
                        in_v, [fvec[c0], c_even])
                for c0 in range(4):
                    out_v[r, pl.ds(64 + c0 * 16, 16)] = plsc.load_gather(
                        in_v, [fvec[c0], c_odd])
                return carry

            lax.fori_loop(0, BLK_COLS // 2, row_body, 0, unroll=2)

        in_dma(jb0, in_a, si_a).start()

        def pair_body(p, carry):
            ja = jb0 + 2 * p
            in_dma(ja + 1, in_b, si_b).start()
            in_dma(ja, in_a, si_a).wait()
            transpose(in_a, o_a)

            @pl.when(p > 0)
            def _():
                out_dma(ja - 2, o_a, so_a).wait()

            out_dma(ja, o_a, so_a).start()

            @pl.when(p + 1 < BLOCKS_PER_W // 2)
            def _():
                in_dma(ja + 2, in_a, si_a).start()

            in_dma(ja + 1, in_b, si_b).wait()
            transpose(in_b, o_b)

            @pl.when(p > 0)
            def _():
                out_dma(ja - 1, o_b, so_b).wait()

            out_dma(ja + 1, o_b, so_b).start()
            return carry

        lax.fori_loop(0, BLOCKS_PER_W // 2, pair_body, 0)
        last = jb0 + BLOCKS_PER_W - 1
        out_dma(last - 1, o_a, so_a).wait()
        out_dma(last, o_b, so_b).wait()

        @pl.when(wid < N_EXTRA)
        def _():
            jb = N_BLOCKS - N_EXTRA + wid
            in_dma(jb, in_a, si_a).start()
            in_dma(jb, in_a, si_a).wait()
            transpose(in_a, o_a)
            out_dma(jb, o_a, so_a).start()
            out_dma(jb, o_a, so_a).wait()

        @pl.when(wid == N_EXTRA)
        def _():
            pltpu.sync_copy(tail_hbm, tbuf)
            pltpu.sync_copy(
                tbuf, out_hbm.at[pl.ds(V_MAIN // 2, V_TAIL // 2), :])

    return k1


@functools.lru_cache(maxsize=None)
def _make_gather(B, V, C):
    b_per_w = B // NUM_WORKERS
    n_chunks = b_per_w // C

    @functools.partial(
        pl.kernel,
        out_type=jax.ShapeDtypeStruct((B, D_MODEL), jnp.float32),
        mesh=_mesh(),
        compiler_params=pltpu.CompilerParams(use_tc_tiling_on_sc=False),
        scratch_types=[
            pltpu.VMEM((C,), jnp.int32),
            pltpu.VMEM((C, D_MODEL), jnp.float32),
            pltpu.SemaphoreType.DMA,
        ],
    )
    def gather_k(idx_hbm, table_hbm, out_hbm, idx_v, rows_v, sem):
        wid = lax.axis_index("s") * NUM_CORES + lax.axis_index("c")
        base = wid * b_per_w

        def chunk_body(i, carry):
            off = pl.multiple_of(base + i * C, 8)
            pltpu.sync_copy(idx_hbm.at[pl.ds(off, C)], idx_v)
            pltpu.async_copy(table_hbm.at[idx_v], rows_v, sem).wait()

            def row_body(r, c2):
                for cc in range(D_MODEL // LANES):
                    sl = pl.ds(cc * LANES, LANES)
                    rows_v[r, sl] = rows_v[r, sl] * SCALE
                return c2

            lax.fori_loop(0, C, row_body, 0)
            pltpu.sync_copy(rows_v, out_hbm.at[pl.ds(off, C)])
            return carry

        lax.fori_loop(0, n_chunks, chunk_body, 0)

    return gather_k


def kernel(x, table):
    B = x.shape[0] * x.shape[1]
    V = table.shape[0]
    xf = x.reshape(B).astype(jnp.int32)
    tail = table[V_MAIN:].reshape(V_TAIL // 2, 128)
    L = _make_transpose()(table.T, tail)
    t_lin = L.reshape(V, D_MODEL)
    out = _make_gather(B, V, 512)(xf, t_lin)
    return out.reshape(x.shape[0], x.shape[1], D_MODEL)


# speedup vs baseline: 3.4901x; 3.4901x over previous
import functools

import numpy as np
import jax
import jax.numpy as jnp
from jax import lax
from jax.experimental import pallas as pl
from jax.experimental.pallas import tpu as pltpu
from jax.experimental.pallas import tpu_sc as plsc

D_MODEL = 64
NUM_CORES = 2
NUM_SUBCORES = 16
NUM_WORKERS = NUM_CORES * NUM_SUBCORES
SCALE = float(np.sqrt(D_MODEL))

V_TOTAL = 1_000_000
V_MAIN = (V_TOTAL // 128) * 128
V_TAIL = V_TOTAL - V_MAIN
BLK_COLS = 256
N_BLOCKS = V_MAIN // BLK_COLS
BLOCKS_PER_W = N_BLOCKS // NUM_WORKERS
N_EXTRA = N_BLOCKS - BLOCKS_PER_W * NUM_WORKERS


def _mesh():
    return plsc.VectorSubcoreMesh(core_axis_name="c", subcore_axis_name="s")


@functools.lru_cache(maxsize=None)
def _make_transpose():
    BLK_WORDS = BLK_COLS * D_MODEL

    @functools.partial(
        pl.kernel,
        out_type=jax.ShapeDtypeStruct((V_TOTAL * D_MODEL,), jnp.float32),
        mesh=_mesh(),
        compiler_params=pltpu.CompilerParams(
            use_tc_tiling_on_sc=True, needs_layout_passes=False),
        scratch_types=[
            pltpu.VMEM((D_MODEL, BLK_COLS), jnp.float32),
            pltpu.VMEM((D_MODEL, BLK_COLS), jnp.float32),
            pltpu.VMEM((BLK_WORDS,), jnp.float32),
            pltpu.VMEM((BLK_WORDS,), jnp.float32),
            pltpu.VMEM((V_TAIL * D_MODEL,), jnp.float32),
            pltpu.SemaphoreType.DMA,
            pltpu.SemaphoreType.DMA,
            pltpu.SemaphoreType.DMA,
            pltpu.SemaphoreType.DMA,
        ],
    )
    def k1(tt_hbm, tail_hbm, out_hbm, in_a, in_b, o_a, o_b, tbuf,
           si_a, si_b, so_a, so_b):
        wid = lax.axis_index("s") * NUM_CORES + lax.axis_index("c")
        jb0 = wid * BLOCKS_PER_W

        iota = lax.iota(jnp.int32, 16)

        def in_dma(jb, buf, sem):
            col = pl.multiple_of(jb * BLK_COLS, BLK_COLS)
            return pltpu.make_async_copy(
                tt_hbm.at[:, pl.ds(col, BLK_COLS)], buf, sem)

        def out_dma(jb, buf, sem):
            off = pl.multiple_of(jb * BLK_WORDS, 8)
            return pltpu.make_async_copy(
                buf, out_hbm.at[pl.ds(off, BLK_WORDS)], sem)

        def transpose(in_v, out_v):
            def v_body(vi, carry):
                vcol = iota + vi * 16
                base = (vcol << 6)
                for f0 in range(0, D_MODEL, 16):
                    for h in range(0, 16, 8):
                        rows = [
                            ((iota + (h + k)) & 15) + f0 for k in range(8)
                        ]
                        gs = [
                            plsc.load_gather(in_v, [rows[k], vcol])
                            for k in range(8)
                        ]
                        for k in range(8):
                            plsc.store_scatter(
                                out_v, [base + rows[k]], gs[k] * SCALE)
                return carry

            lax.fori_loop(0, BLK_COLS // 16, v_body, 0)

        in_dma(jb0, in_a, si_a).start()

        def pair_body(p, carry):
            ja = jb0 + 2 * p
            in_dma(ja + 1, in_b, si_b).start()
            in_dma(ja, in_a, si_a).wait()
            transpose(in_a, o_a)

            @pl.when(p > 0)
            def _():
                out_dma(ja - 2, o_a, so_a).wait()

            out_dma(ja, o_a, so_a).start()

            @pl.when(p + 1 < BLOCKS_PER_W // 2)
            def _():
                in_dma(ja + 2, in_a, si_a).start()

            in_dma(ja + 1, in_b, si_b).wait()
            transpose(in_b, o_b)

            @pl.when(p > 0)
            def _():
                out_dma(ja - 1, o_b, so_b).wait()

            out_dma(ja + 1, o_b, so_b).start()
            return carry

        lax.fori_loop(0, BLOCKS_PER_W // 2, pair_body, 0)
        last = jb0 + BLOCKS_PER_W - 1
        out_dma(last - 1, o_a, so_a).wait()
        out_dma(last, o_b, so_b).wait()

        @pl.when(wid < N_EXTRA)
        def _():
            jb = N_BLOCKS - N_EXTRA + wid
            in_dma(jb, in_a, si_a).start()
            in_dma(jb, in_a, si_a).wait()
            transpose(in_a, o_a)
            out_dma(jb, o_a, so_a).start()
            out_dma(jb, o_a, so_a).wait()

        @pl.when(wid == N_EXTRA)
        def _():
            pltpu.sync_copy(tail_hbm, tbuf)
            pltpu.sync_copy(
                tbuf,
                out_hbm.at[pl.ds(V_MAIN * D_MODEL, V_TAIL * D_MODEL)])

    return k1


@functools.lru_cache(maxsize=None)
def _make_gather(B, V, C):
    b_per_w = B // NUM_WORKERS
    n_chunks = b_per_w // C

    @functools.partial(
        pl.kernel,
        out_type=jax.ShapeDtypeStruct((B, D_MODEL), jnp.float32),
        mesh=_mesh(),
        compiler_params=pltpu.CompilerParams(use_tc_tiling_on_sc=False),
        scratch_types=[
            pltpu.VMEM((C,), jnp.int32),
            pltpu.VMEM((C,), jnp.int32),
            pltpu.VMEM((C, D_MODEL), jnp.float32),
            pltpu.VMEM((C, D_MODEL), jnp.float32),
            pltpu.SemaphoreType.DMA,
            pltpu.SemaphoreType.DMA,
            pltpu.SemaphoreType.DMA,
            pltpu.SemaphoreType.DMA,
        ],
    )
    def gather_k(idx_hbm, table_hbm, out_hbm, idx_a, idx_b, rows_a, rows_b,
                 ga, gb, oa, ob):
        wid = lax.axis_index("s") * NUM_CORES + lax.axis_index("c")
        base = wid * b_per_w

        def start_gather(j, idx_v, rows_v, gsem):
            off = pl.multiple_of(base + j * C, 8)
            pltpu.sync_copy(idx_hbm.at[pl.ds(off, C)], idx_v)
            pltpu.make_async_copy(table_hbm.at[idx_v], rows_v, gsem).start()

        def wait_gather(idx_v, rows_v, gsem):
            pltpu.make_async_copy(table_hbm.at[idx_v], rows_v, gsem).wait()

        def out(j, rows_v, osem):
            off = pl.multiple_of(base + j * C, 8)
            return pltpu.make_async_copy(
                rows_v, out_hbm.at[pl.ds(off, C)], osem)

        start_gather(0, idx_a, rows_a, ga)

        def pair_body(p, carry):
            ja = 2 * p
            jb = ja + 1

            @pl.when(p > 0)
            def _():
                out(jb - 2, rows_b, ob).wait()

            start_gather(jb, idx_b, rows_b, gb)
            wait_gather(idx_a, rows_a, ga)
            out(ja, rows_a, oa).start()
            wait_gather(idx_b, rows_b, gb)
            out(jb, rows_b, ob).start()

            @pl.when(p + 1 < n_chunks // 2)
            def _():
                out(ja, rows_a, oa).wait()
                start_gather(ja + 2, idx_a, rows_a, ga)

            return carry

        lax.fori_loop(0, n_chunks // 2, pair_body, 0)
        out(n_chunks - 2, rows_a, oa).wait()
        out(n_chunks - 1, rows_b, ob).wait()

    return gather_k


def kernel(x, table):
    B = x.shape[0] * x.shape[1]
    V = table.shape[0]
    xf = x.reshape(B).astype(jnp.int32)
    tail = table[V_MAIN:].reshape(-1) * SCALE
    L = _make_transpose()(table.T, tail)
    t_lin = L.reshape(V, D_MODEL)
    out = _make_gather(B, V, 640)(xf, t_lin)
    return out.reshape(x.shape[0], x.shape[1], D_MODEL)
